# baseline (speedup 1.0000x reference)
import functools

import jax
import jax.numpy as jnp
from jax import lax
from jax.experimental import pallas as pl
from jax.experimental.pallas import tpu as pltpu
from jax.experimental.pallas import tpu_sc as plsc

N = 100000
K = 128
GROUP = 32
NG = K // GROUP
B = 16384

NC = 2
NS = 16
NW = NC * NS
BPW = B // NW
CHUNK = 64
NCHUNK = BPW // CHUNK
LANES = 16


def _sc_kernel(x_hbm, q_hbm, s_hbm, out_hbm, idx_v, q_v, sidx_v, s4_v, o_v,
               gsem0, gsem1, wsem0, wsem1):
    wid = lax.axis_index("s") * NC + lax.axis_index("c")
    base = wid * BPW
    gsems = (gsem0, gsem1)
    wsems = (wsem0, wsem1)

    pltpu.sync_copy(x_hbm.at[pl.ds(base, BPW)], idx_v)

    def build_sidx(c, b):
        for t in range(CHUNK // LANES):
            tok = idx_v[pl.ds(c * CHUNK + t * LANES, LANES)]
            for g in range(NG):
                sidx_v[b, g, pl.ds(t * LANES, LANES)] = tok + (g * N)

    def fire(c, b):
        pltpu.async_copy(q_hbm.at[idx_v.at[pl.ds(c * CHUNK, CHUNK)]],
                         q_v.at[b], gsems[b])
        for g in range(NG):
            pltpu.async_copy(s_hbm.at[sidx_v.at[b, g]],
                             s4_v.at[b, pl.ds(g * CHUNK, CHUNK)], gsems[b])

    def compute(b):
        @plsc.parallel_loop(0, CHUNK // LANES, 1, unroll=2)
        def blk_body(t):
            rbase = t * LANES
            for g in range(NG):
                s16 = s4_v[b, pl.ds(g * CHUNK + rbase, LANES)]
                for l in range(LANES):
                    lidx = jnp.full((LANES,), l, jnp.int32)
                    svec = jnp.take_along_axis(
                        s16, lidx, axis=0, mode="promise_in_bounds")
                    r = rbase + l
                    for h in range(2):
                        j = g * 2 + h
                        q16 = q_v[b, r, pl.ds(j * LANES, LANES)]
                        o_v[b, r, pl.ds(j * LANES, LANES)] = (
                            q16.astype(jnp.float32) * svec)

    def drain_gather(b):
        pltpu.make_async_copy(q_hbm.at[pl.ds(0, CHUNK)], q_v.at[b],
                              gsems[b]).wait()
        pltpu.make_async_copy(s_hbm.at[pl.ds(0, NG * CHUNK)], s4_v.at[b],
                              gsems[b]).wait()

    def drain_wb(b):
        pltpu.make_async_copy(o_v.at[b], out_hbm.at[pl.ds(0, CHUNK)],
                              wsems[b]).wait()

    def half(i, c, b):
        nxt = c + 2
        drain_gather(b)
        @pl.when(i > 0)
        def _():
            drain_wb(b)
        compute(b)
        pltpu.async_copy(
            o_v.at[b], out_hbm.at[pl.ds(base + c * CHUNK, CHUNK)], wsems[b])
        @pl.when(nxt < NCHUNK)
        def _():
            build_sidx(nxt, b)
            fire(nxt, b)

    build_sidx(0, 0)
    fire(0, 0)
    build_sidx(1, 1)
    fire(1, 1)

    def pair(i, carry):
        c0 = 2 * i
        half(i, c0, 0)
        half(i, c0 + 1, 1)
        return carry

    lax.fori_loop(0, NCHUNK // 2, pair, 0)
    drain_wb(0)
    drain_wb(1)


def kernel(x, qweights, scales):
    mesh = plsc.VectorSubcoreMesh(core_axis_name="c", subcore_axis_name="s")
    run = functools.partial(
        pl.kernel,
        mesh=mesh,
        out_type=jax.ShapeDtypeStruct((B, K), jnp.float32),
        scratch_types=[
            pltpu.VMEM((BPW,), jnp.int32),
            pltpu.VMEM((2, CHUNK, K), jnp.int32),
            pltpu.VMEM((2, NG, CHUNK), jnp.int32),
            pltpu.VMEM((2, NG * CHUNK), jnp.float32),
            pltpu.VMEM((2, CHUNK, K), jnp.float32),
            pltpu.SemaphoreType.DMA,
            pltpu.SemaphoreType.DMA,
            pltpu.SemaphoreType.DMA,
            pltpu.SemaphoreType.DMA,
        ],
    )(_sc_kernel)
    return run(x, qweights, scales.T.reshape(N * NG))


# speedup vs baseline: 1.1038x; 1.1038x over previous
import functools

import jax
import jax.numpy as jnp
from jax import lax
from jax.experimental import pallas as pl
from jax.experimental.pallas import tpu as pltpu
from jax.experimental.pallas import tpu_sc as plsc

N = 100000
K = 128
GROUP = 32
NG = K // GROUP
B = 16384

NC = 2
NS = 16
NW = NC * NS
BPW = B // NW
CHUNK = 128
NCHUNK = BPW // CHUNK
LANES = 16


def _sc_kernel(x_hbm, q_hbm, s_hbm, out_hbm, idx_v, q_v, sidx_v, s4_v, o_v,
               gsem, wsem):
    wid = lax.axis_index("s") * NC + lax.axis_index("c")
    base = wid * BPW

    pltpu.sync_copy(x_hbm.at[pl.ds(base, BPW)], idx_v)

    def build_sidx(c, b):
        for t in range(CHUNK // LANES):
            tok = idx_v[pl.ds(c * CHUNK + t * LANES, LANES)]
            for g in range(NG):
                sidx_v[b, g, pl.ds(t * LANES, LANES)] = tok + (g * N)

    def fire(c, b):
        pltpu.async_copy(q_hbm.at[idx_v.at[pl.ds(c * CHUNK, CHUNK)]],
                         q_v.at[b], gsem.at[b])
        for g in range(NG):
            pltpu.async_copy(s_hbm.at[sidx_v.at[b, g]],
                             s4_v.at[b, pl.ds(g * CHUNK, CHUNK)], gsem.at[b])

    def compute(b):
        @plsc.parallel_loop(0, CHUNK // LANES, 1, unroll=2)
        def blk_body(t):
            rbase = t * LANES
            for g in range(NG):
                s16 = s4_v[b, pl.ds(g * CHUNK + rbase, LANES)]
                for l in range(LANES):
                    lidx = jnp.full((LANES,), l, jnp.int32)
                    svec = jnp.take_along_axis(
                        s16, lidx, axis=0, mode="promise_in_bounds")
                    r = rbase + l
                    for h in range(2):
                        j = g * 2 + h
                        q16 = q_v[b, r, pl.ds(j * LANES, LANES)]
                        o_v[b, r, pl.ds(j * LANES, LANES)] = (
                            q16.astype(jnp.float32) * svec)

    def drain_gather(b):
        pltpu.make_async_copy(q_hbm.at[pl.ds(0, CHUNK)], q_v.at[b],
                              gsem.at[b]).wait()
        pltpu.make_async_copy(s_hbm.at[pl.ds(0, NG * CHUNK)], s4_v.at[b],
                              gsem.at[b]).wait()

    def drain_wb(b):
        pltpu.make_async_copy(o_v.at[b], out_hbm.at[pl.ds(0, CHUNK)],
                              wsem.at[b]).wait()

    build_sidx(0, 0)
    fire(0, 0)
    build_sidx(1, 1)
    fire(1, 1)

    def body(c, carry):
        b = lax.rem(c, 2)
        drain_gather(b)
        @pl.when(c >= 2)
        def _():
            drain_wb(b)
        compute(b)
        pltpu.async_copy(
            o_v.at[b], out_hbm.at[pl.ds(base + c * CHUNK, CHUNK)], wsem.at[b])
        @pl.when(c + 2 < NCHUNK)
        def _():
            build_sidx(c + 2, b)
            fire(c + 2, b)
        return carry

    lax.fori_loop(0, NCHUNK, body, 0)
    drain_wb(0)
    drain_wb(1)


def kernel(x, qweights, scales):
    mesh = plsc.VectorSubcoreMesh(core_axis_name="c", subcore_axis_name="s")
    run = functools.partial(
        pl.kernel,
        mesh=mesh,
        out_type=jax.ShapeDtypeStruct((B, K), jnp.float32),
        scratch_types=[
            pltpu.VMEM((BPW,), jnp.int32),
            pltpu.VMEM((2, CHUNK, K), jnp.int32),
            pltpu.VMEM((2, NG, CHUNK), jnp.int32),
            pltpu.VMEM((2, NG * CHUNK), jnp.float32),
            pltpu.VMEM((2, CHUNK, K), jnp.float32),
            pltpu.SemaphoreType.DMA((2,)),
            pltpu.SemaphoreType.DMA((2,)),
        ],
    )(_sc_kernel)
    return run(x, qweights, scales.T.reshape(N * NG))
